# baseline (speedup 1.0000x reference)
import functools

import jax
import jax.numpy as jnp
from jax.experimental import pallas as pl
from jax.experimental.pallas import tpu as pltpu


def _round_up(x, m):
    return (x + m - 1) // m * m


def _fused_kernel(a_ref, v_ref, s_ref, w1, w2, b2, w3, b3, pw4, pb4,
                  fw4, fb4, o_ref, *, n_inv, rank, h3_split):
    for j in range(a_ref.shape[0]):
        _one_batch(a_ref[j], v_ref[j], s_ref[j], w1, w2, b2, w3, b3,
                   pw4, pb4, fw4, fb4, o_ref.at[j], n_inv, rank, h3_split)


def _one_batch(x, vv, s, w1, w2, b2, w3, b3, pw4, pb4, fw4, fb4,
               o_ref, n_inv, rank, h3_split):
    bf16 = jnp.bfloat16
    h = jnp.dot(x, w1[...], preferred_element_type=jnp.float32)
    h = jnp.maximum(h.astype(bf16), 0)
    h = jnp.dot(h, w2[...], preferred_element_type=jnp.float32) + b2[...]
    h = jnp.maximum(h.astype(bf16), 0)
    h = jnp.dot(h, w3[...], preferred_element_type=jnp.float32) + b3[...]
    h = jnp.maximum(h.astype(bf16), 0)
    h3p = h[:, :h3_split]
    h3f = h[:, h3_split:]
    c = jax.lax.dot_general(h3p, vv, (((0,), (0,)), ((), ())),
                            preferred_element_type=jnp.float32)
    u_dt = jax.lax.dot_general(pw4[...], c, (((0,), (0,)), ((), ())),
                               preferred_element_type=jnp.float32)
    u_dt = u_dt + pb4[...].reshape(-1, 1) * s
    d_dim, i_dim = u_dt.shape
    o_dim = o_ref.shape[-1]
    drow = jax.lax.broadcasted_iota(jnp.int32, (d_dim, i_dim), 0)
    icol = jax.lax.broadcasted_iota(jnp.int32, (d_dim, i_dim), 1)
    diag = jnp.where((drow % (i_dim * rank)) // rank == icol, u_dt, 0.0)
    u = jnp.sum(diag, axis=1, keepdims=True)
    blk = jax.lax.broadcasted_iota(jnp.int32, (d_dim, o_dim), 0) // (
        d_dim // o_dim)
    oix = jax.lax.broadcasted_iota(jnp.int32, (d_dim, o_dim), 1)
    su = jnp.where(blk == oix, u * n_inv, 0.0)
    w4_eff = jnp.dot(fw4[...], su, preferred_element_type=jnp.float32)
    b4_eff = jnp.dot(fb4[...], su, preferred_element_type=jnp.float32)
    out = jnp.dot(h3f, w4_eff.astype(bf16),
                  preferred_element_type=jnp.float32) + b4_eff
    o_ref[...] = out.astype(o_ref.dtype)


def _full_spec(p):
    return pl.BlockSpec(p.shape, lambda b: (0, 0))


def _block_diag(a, b):
    (ka, na), (kb, nb) = a.shape, b.shape
    return jnp.concatenate([
        jnp.concatenate([a, jnp.zeros((ka, nb), a.dtype)], axis=1),
        jnp.concatenate([jnp.zeros((kb, na), b.dtype), b], axis=1)], axis=0)


def kernel(v, a, psi_w0, psi_b0, psi_w1, psi_b1, psi_w2, psi_b2, psi_w3,
           psi_b3, phi_w0, phi_b0, phi_w1, phi_b1, phi_w2, phi_b2, phi_w3,
           phi_b3):
    B, N, I = v.shape
    D = psi_w3.shape[1]
    O = I
    R = D // (O * I)

    n_pad = _round_up(N, 8)
    if n_pad != N:
        a_p = jnp.pad(a, ((0, 0), (0, n_pad - N), (0, 0)))
        v_p = jnp.pad(v, ((0, 0), (0, n_pad - N), (0, 0)))
    else:
        a_p, v_p = a, v

    bf16 = jnp.bfloat16
    a_p = jnp.concatenate(
        [a_p, jnp.ones(a_p.shape[:2] + (1,), a_p.dtype)], axis=2)
    a_p = a_p.astype(bf16)
    v_b = v_p.astype(bf16)
    s_all = jnp.sum(v_b.astype(jnp.float32), axis=1, keepdims=True)

    w1 = jnp.concatenate(
        [jnp.concatenate([psi_w0, phi_w0], axis=1),
         jnp.concatenate([psi_b0, phi_b0], axis=1)], axis=0).astype(bf16)
    w2 = _block_diag(psi_w1, phi_w1).astype(bf16)
    b2 = jnp.concatenate([psi_b1, phi_b1], axis=1)
    w3 = _block_diag(psi_w2, phi_w2).astype(bf16)
    b3 = jnp.concatenate([psi_b2, phi_b2], axis=1)

    params = [w1, w2, b2, w3, b3, psi_w3, psi_b3, phi_w3, phi_b3]

    G = 4 if B % 4 == 0 else (2 if B % 2 == 0 else 1)
    out_pad = pl.pallas_call(
        functools.partial(_fused_kernel, n_inv=1.0 / float(N), rank=R,
                          h3_split=psi_w2.shape[1]),
        grid=(B // G,),
        in_specs=[pl.BlockSpec((G, n_pad, 4), lambda b: (b, 0, 0)),
                  pl.BlockSpec((G, n_pad, I), lambda b: (b, 0, 0)),
                  pl.BlockSpec((G, 1, I), lambda b: (b, 0, 0))]
                 + [_full_spec(p) for p in params],
        out_specs=pl.BlockSpec((G, n_pad, O), lambda b: (b, 0, 0)),
        out_shape=jax.ShapeDtypeStruct((B, n_pad, O), v.dtype),
        compiler_params=pltpu.CompilerParams(
            dimension_semantics=("parallel",)),
    )(a_p, v_b, s_all, *params)

    return out_pad[:, :N, :]


# speedup vs baseline: 1.0998x; 1.0998x over previous
import functools

import jax
import jax.numpy as jnp
from jax.experimental import pallas as pl
from jax.experimental.pallas import tpu as pltpu


def _round_up(x, m):
    return (x + m - 1) // m * m


def _fused_kernel(a_ref, v_ref, s_ref, w1, b1, w2, b2, w3, b3, pw4, pb4,
                  fw4, fb4, o_ref, *, n_inv, rank, h3_split):
    for j in range(a_ref.shape[0]):
        _one_batch(a_ref[j], v_ref[j], s_ref[j], w1, b1, w2, b2, w3, b3,
                   pw4, pb4, fw4, fb4, o_ref.at[j], n_inv, rank, h3_split)


def _one_batch(x, vv, s, w1, b1, w2, b2, w3, b3, pw4, pb4, fw4, fb4,
               o_ref, n_inv, rank, h3_split):
    bf16 = jnp.bfloat16
    h = jnp.dot(x, w1[...], preferred_element_type=jnp.float32) + b1[...]
    h = jnp.maximum(h.astype(bf16), 0)
    h = jnp.dot(h, w2[...], preferred_element_type=jnp.float32) + b2[...]
    h = jnp.maximum(h.astype(bf16), 0)
    h = jnp.dot(h, w3[...], preferred_element_type=jnp.float32) + b3[...]
    h = jnp.maximum(h.astype(bf16), 0)
    h3p = h[:, :h3_split]
    h3f = h[:, h3_split:]
    c = jax.lax.dot_general(h3p, vv, (((0,), (0,)), ((), ())),
                            preferred_element_type=jnp.float32)
    u_dt = jax.lax.dot_general(pw4[...], c, (((0,), (0,)), ((), ())),
                               preferred_element_type=jnp.float32)
    u_dt = u_dt + pb4[...].reshape(-1, 1) * s
    d_dim, i_dim = u_dt.shape
    o_dim = o_ref.shape[-1]
    drow = jax.lax.broadcasted_iota(jnp.int32, (d_dim, i_dim), 0)
    icol = jax.lax.broadcasted_iota(jnp.int32, (d_dim, i_dim), 1)
    diag = jnp.where((drow % (i_dim * rank)) // rank == icol, u_dt, 0.0)
    u = jnp.sum(diag, axis=1, keepdims=True)
    blk = jax.lax.broadcasted_iota(jnp.int32, (d_dim, o_dim), 0) // (
        d_dim // o_dim)
    oix = jax.lax.broadcasted_iota(jnp.int32, (d_dim, o_dim), 1)
    su = jnp.where(blk == oix, u * n_inv, 0.0)
    w4_eff = jnp.dot(fw4[...], su, preferred_element_type=jnp.float32)
    b4_eff = jnp.dot(fb4[...], su, preferred_element_type=jnp.float32)
    out = jnp.dot(h3f, w4_eff.astype(bf16),
                  preferred_element_type=jnp.float32) + b4_eff
    o_ref[...] = out.astype(o_ref.dtype)


def _full_spec(p):
    return pl.BlockSpec(p.shape, lambda b: (0, 0))


def _block_diag(a, b):
    (ka, na), (kb, nb) = a.shape, b.shape
    return jnp.concatenate([
        jnp.concatenate([a, jnp.zeros((ka, nb), a.dtype)], axis=1),
        jnp.concatenate([jnp.zeros((kb, na), b.dtype), b], axis=1)], axis=0)


def kernel(v, a, psi_w0, psi_b0, psi_w1, psi_b1, psi_w2, psi_b2, psi_w3,
           psi_b3, phi_w0, phi_b0, phi_w1, phi_b1, phi_w2, phi_b2, phi_w3,
           phi_b3):
    B, N, I = v.shape
    D = psi_w3.shape[1]
    O = I
    R = D // (O * I)

    n_pad = _round_up(N, 8)
    if n_pad != N:
        a_p = jnp.pad(a, ((0, 0), (0, n_pad - N), (0, 0)))
        v_p = jnp.pad(v, ((0, 0), (0, n_pad - N), (0, 0)))
    else:
        a_p, v_p = a, v

    bf16 = jnp.bfloat16
    a_p = a_p.astype(bf16)
    v_b = v_p.astype(bf16)
    s_all = jnp.sum(v_b.astype(jnp.float32), axis=1, keepdims=True)

    w1 = jnp.concatenate([psi_w0, phi_w0], axis=1).astype(bf16)
    b1 = jnp.concatenate([psi_b0, phi_b0], axis=1)
    w2 = _block_diag(psi_w1, phi_w1).astype(bf16)
    b2 = jnp.concatenate([psi_b1, phi_b1], axis=1)
    w3 = _block_diag(psi_w2, phi_w2).astype(bf16)
    b3 = jnp.concatenate([psi_b2, phi_b2], axis=1)

    params = [w1, b1, w2, b2, w3, b3, psi_w3, psi_b3, phi_w3, phi_b3]

    G = 4 if B % 4 == 0 else (2 if B % 2 == 0 else 1)
    out_pad = pl.pallas_call(
        functools.partial(_fused_kernel, n_inv=1.0 / float(N), rank=R,
                          h3_split=psi_w2.shape[1]),
        grid=(B // G,),
        in_specs=[pl.BlockSpec((G, n_pad, 3), lambda b: (b, 0, 0)),
                  pl.BlockSpec((G, n_pad, I), lambda b: (b, 0, 0)),
                  pl.BlockSpec((G, 1, I), lambda b: (b, 0, 0))]
                 + [_full_spec(p) for p in params],
        out_specs=pl.BlockSpec((G, n_pad, O), lambda b: (b, 0, 0)),
        out_shape=jax.ShapeDtypeStruct((B, n_pad, O), v.dtype),
        compiler_params=pltpu.CompilerParams(
            dimension_semantics=("parallel",)),
    )(a_p, v_b, s_all, *params)

    return out_pad[:, :N, :]


# speedup vs baseline: 1.6556x; 1.5054x over previous
import functools

import jax
import jax.numpy as jnp
from jax.experimental import pallas as pl
from jax.experimental.pallas import tpu as pltpu


def _round_up(x, m):
    return (x + m - 1) // m * m


def _fused_kernel(a_ref, v_ref, s_ref, w1, b1, w2, b2, w3, b3, pw4, pb4,
                  fw4, fb4, o_ref, *, n_inv, rank, h3_split):
    for j in range(a_ref.shape[0]):
        _one_batch(a_ref[j], v_ref[j], s_ref[j], w1, b1, w2, b2, w3, b3,
                   pw4, pb4, fw4, fb4, o_ref.at[j], n_inv, rank, h3_split)


def _one_batch(xt, vt, s, w1, b1, w2, b2, w3, b3, pw4, pb4, fw4, fb4,
               o_ref, n_inv, rank, h3_split):
    bf16 = jnp.bfloat16
    h = jnp.dot(w1[...], xt, preferred_element_type=jnp.float32) + b1[...]
    h = jnp.maximum(h.astype(bf16), 0)
    h = jnp.dot(w2[...], h, preferred_element_type=jnp.float32) + b2[...]
    h = jnp.maximum(h.astype(bf16), 0)
    h = jnp.dot(w3[...], h, preferred_element_type=jnp.float32) + b3[...]
    h = jnp.maximum(h.astype(bf16), 0)
    h3p = h[:h3_split, :]
    h3f = h[h3_split:, :]
    c = jax.lax.dot_general(h3p, vt, (((1,), (1,)), ((), ())),
                            preferred_element_type=jnp.float32)
    u_dt = jax.lax.dot_general(pw4[...], c, (((0,), (0,)), ((), ())),
                               preferred_element_type=jnp.float32)
    u_dt = u_dt + pb4[...].reshape(-1, 1) * s
    d_dim, i_dim = u_dt.shape
    o_dim = o_ref.shape[0]
    drow = jax.lax.broadcasted_iota(jnp.int32, (d_dim, i_dim), 0)
    icol = jax.lax.broadcasted_iota(jnp.int32, (d_dim, i_dim), 1)
    diag = jnp.where((drow % (i_dim * rank)) // rank == icol, u_dt, 0.0)
    u = jnp.sum(diag, axis=1, keepdims=True)
    blk = jax.lax.broadcasted_iota(jnp.int32, (d_dim, o_dim), 0) // (
        d_dim // o_dim)
    oix = jax.lax.broadcasted_iota(jnp.int32, (d_dim, o_dim), 1)
    su = jnp.where(blk == oix, u * n_inv, 0.0)
    w4_eff = jnp.dot(fw4[...], su, preferred_element_type=jnp.float32)
    b4_eff = jax.lax.dot_general(su, fb4[...], (((0,), (1,)), ((), ())),
                                 preferred_element_type=jnp.float32)
    out = jax.lax.dot_general(w4_eff.astype(bf16), h3f,
                              (((0,), (0,)), ((), ())),
                              preferred_element_type=jnp.float32) + b4_eff
    o_ref[...] = out.astype(o_ref.dtype)


def _full_spec2(p):
    return pl.BlockSpec(p.shape, lambda b: (0, 0))


def _block_diag(a, b):
    (ka, na), (kb, nb) = a.shape, b.shape
    return jnp.concatenate([
        jnp.concatenate([a, jnp.zeros((ka, nb), a.dtype)], axis=1),
        jnp.concatenate([jnp.zeros((kb, na), b.dtype), b], axis=1)], axis=0)


def kernel(v, a, psi_w0, psi_b0, psi_w1, psi_b1, psi_w2, psi_b2, psi_w3,
           psi_b3, phi_w0, phi_b0, phi_w1, phi_b1, phi_w2, phi_b2, phi_w3,
           phi_b3):
    B, N, I = v.shape
    D = psi_w3.shape[1]
    O = I
    R = D // (O * I)

    n_pad = _round_up(N, 8)
    if n_pad != N:
        a_p = jnp.pad(a, ((0, 0), (0, n_pad - N), (0, 0)))
        v_p = jnp.pad(v, ((0, 0), (0, n_pad - N), (0, 0)))
    else:
        a_p, v_p = a, v

    bf16 = jnp.bfloat16
    at = jnp.swapaxes(a_p, 1, 2).astype(bf16)
    vt = jnp.swapaxes(v_p, 1, 2).astype(bf16)
    s_all = jnp.sum(vt.astype(jnp.float32), axis=2)[:, None, :]

    w1 = jnp.concatenate([psi_w0, phi_w0], axis=1).T.astype(bf16)
    b1 = jnp.concatenate([psi_b0, phi_b0], axis=1).T
    w2 = _block_diag(psi_w1, phi_w1).T.astype(bf16)
    b2 = jnp.concatenate([psi_b1, phi_b1], axis=1).T
    w3 = _block_diag(psi_w2, phi_w2).T.astype(bf16)
    b3 = jnp.concatenate([psi_b2, phi_b2], axis=1).T

    params = [w1, b1, w2, b2, w3, b3, psi_w3, psi_b3, phi_w3, phi_b3]

    G = 4 if B % 4 == 0 else (2 if B % 2 == 0 else 1)
    out_t = pl.pallas_call(
        functools.partial(_fused_kernel, n_inv=1.0 / float(N), rank=R,
                          h3_split=psi_w2.shape[1]),
        grid=(B // G,),
        in_specs=[pl.BlockSpec((G, 3, n_pad), lambda b: (b, 0, 0)),
                  pl.BlockSpec((G, I, n_pad), lambda b: (b, 0, 0)),
                  pl.BlockSpec((G, 1, I), lambda b: (b, 0, 0))]
                 + [_full_spec2(p) for p in params],
        out_specs=pl.BlockSpec((G, O, n_pad), lambda b: (b, 0, 0)),
        out_shape=jax.ShapeDtypeStruct((B, O, n_pad), v.dtype),
        compiler_params=pltpu.CompilerParams(
            dimension_semantics=("parallel",)),
    )(at, vt, s_all, *params)

    return jnp.swapaxes(out_t, 1, 2)[:, :N, :]


# speedup vs baseline: 1.6653x; 1.0058x over previous
import functools

import jax
import jax.numpy as jnp
from jax.experimental import pallas as pl
from jax.experimental.pallas import tpu as pltpu


def _round_up(x, m):
    return (x + m - 1) // m * m


def _fused_kernel(a_ref, v_ref, s_ref, w1, b1, w2, b2, w3, b3, pw4, pb4,
                  fw4, fb4, o_ref, *, n_inv, rank, h3_split):
    for j in range(a_ref.shape[0]):
        _one_batch(a_ref[j], v_ref[j], s_ref[j], w1, b1, w2, b2, w3, b3,
                   pw4, pb4, fw4, fb4, o_ref.at[j], n_inv, rank, h3_split)


def _one_batch(xt, vt, s, w1, b1, w2, b2, w3, b3, pw4, pb4, fw4, fb4,
               o_ref, n_inv, rank, h3_split):
    bf16 = jnp.bfloat16
    h = jnp.dot(w1[...], xt, preferred_element_type=jnp.float32) + b1[...]
    h = jnp.maximum(h.astype(bf16), 0)
    h = jnp.dot(w2[...], h, preferred_element_type=jnp.float32) + b2[...]
    h = jnp.maximum(h.astype(bf16), 0)
    h = jnp.dot(w3[...], h, preferred_element_type=jnp.float32) + b3[...]
    h = jnp.maximum(h.astype(bf16), 0)
    h3p = h[:h3_split, :]
    h3f = h[h3_split:, :]
    c = jax.lax.dot_general(h3p, vt, (((1,), (1,)), ((), ())),
                            preferred_element_type=jnp.float32)
    u_dt = jax.lax.dot_general(pw4[...], c, (((0,), (0,)), ((), ())),
                               preferred_element_type=jnp.float32)
    u_dt = u_dt + pb4[...].reshape(-1, 1) * s
    d_dim, i_dim = u_dt.shape
    o_dim = o_ref.shape[0]
    drow = jax.lax.broadcasted_iota(jnp.int32, (d_dim, i_dim), 0)
    icol = jax.lax.broadcasted_iota(jnp.int32, (d_dim, i_dim), 1)
    diag = jnp.where((drow % (i_dim * rank)) // rank == icol, u_dt, 0.0)
    u = jnp.sum(diag, axis=1, keepdims=True)
    blk = jax.lax.broadcasted_iota(jnp.int32, (d_dim, o_dim), 0) // (
        d_dim // o_dim)
    oix = jax.lax.broadcasted_iota(jnp.int32, (d_dim, o_dim), 1)
    su = jnp.where(blk == oix, u * n_inv, 0.0)
    w4_eff = jnp.dot(fw4[...], su, preferred_element_type=jnp.float32)
    b4_eff = jax.lax.dot_general(su, fb4[...], (((0,), (1,)), ((), ())),
                                 preferred_element_type=jnp.float32)
    out = jax.lax.dot_general(w4_eff.astype(bf16), h3f,
                              (((0,), (0,)), ((), ())),
                              preferred_element_type=jnp.float32) + b4_eff
    o_ref[...] = out.astype(o_ref.dtype)


def _full_spec2(p):
    return pl.BlockSpec(p.shape, lambda b: (0, 0))


def _block_diag(a, b):
    (ka, na), (kb, nb) = a.shape, b.shape
    return jnp.concatenate([
        jnp.concatenate([a, jnp.zeros((ka, nb), a.dtype)], axis=1),
        jnp.concatenate([jnp.zeros((kb, na), b.dtype), b], axis=1)], axis=0)


def kernel(v, a, psi_w0, psi_b0, psi_w1, psi_b1, psi_w2, psi_b2, psi_w3,
           psi_b3, phi_w0, phi_b0, phi_w1, phi_b1, phi_w2, phi_b2, phi_w3,
           phi_b3):
    B, N, I = v.shape
    D = psi_w3.shape[1]
    O = I
    R = D // (O * I)

    n_pad = _round_up(N, 8)
    if n_pad != N:
        a_p = jnp.pad(a, ((0, 0), (0, n_pad - N), (0, 0)))
        v_p = jnp.pad(v, ((0, 0), (0, n_pad - N), (0, 0)))
    else:
        a_p, v_p = a, v

    bf16 = jnp.bfloat16
    at = jnp.swapaxes(a_p, 1, 2).astype(bf16)
    vt = jnp.swapaxes(v_p, 1, 2).astype(bf16)
    s_all = jnp.sum(vt.astype(jnp.float32), axis=2)[:, None, :]

    w1 = jnp.concatenate([psi_w0, phi_w0], axis=1).T.astype(bf16)
    b1 = jnp.concatenate([psi_b0, phi_b0], axis=1).T
    w2 = _block_diag(psi_w1, phi_w1).T.astype(bf16)
    b2 = jnp.concatenate([psi_b1, phi_b1], axis=1).T
    w3 = _block_diag(psi_w2, phi_w2).T.astype(bf16)
    b3 = jnp.concatenate([psi_b2, phi_b2], axis=1).T

    params = [w1, b1, w2, b2, w3, b3, psi_w3, psi_b3, phi_w3, phi_b3]

    G = 8 if B % 8 == 0 else (2 if B % 2 == 0 else 1)
    out_t = pl.pallas_call(
        functools.partial(_fused_kernel, n_inv=1.0 / float(N), rank=R,
                          h3_split=psi_w2.shape[1]),
        grid=(B // G,),
        in_specs=[pl.BlockSpec((G, 3, n_pad), lambda b: (b, 0, 0)),
                  pl.BlockSpec((G, I, n_pad), lambda b: (b, 0, 0)),
                  pl.BlockSpec((G, 1, I), lambda b: (b, 0, 0))]
                 + [_full_spec2(p) for p in params],
        out_specs=pl.BlockSpec((G, O, n_pad), lambda b: (b, 0, 0)),
        out_shape=jax.ShapeDtypeStruct((B, O, n_pad), v.dtype),
        compiler_params=pltpu.CompilerParams(
            dimension_semantics=("parallel",)),
    )(at, vt, s_all, *params)

    return jnp.swapaxes(out_t, 1, 2)[:, :N, :]
